# Initial kernel scaffold; baseline (speedup 1.0000x reference)
import jax
import jax.numpy as jnp
from jax.experimental import pallas as pl


def kernel(X):
    raise NotImplementedError("write your pallas kernel here")


# speedup vs baseline: 16.6028x; 16.6028x over previous
import jax
import jax.numpy as jnp
from jax.experimental import pallas as pl
from jax.experimental.pallas import tpu as pltpu

_B = 2
_N = 2048
_D = 128
_K = 16
_RT = 256
_NT = _N // _RT


def _dim_body(x_ref, xt_ref, out_ref, d2_ref):
    for b in range(_B):
        xt = xt_ref[b]
        sq_all = jnp.sum(xt * xt, axis=0, keepdims=True)

        def tile_step(t, total):
            xr = x_ref[b, pl.ds(t * _RT, _RT), :]
            sq_r = jnp.sum(xr * xr, axis=1, keepdims=True)
            g = jax.lax.dot_general(
                xr, xt, (((1,), (0,)), ((), ())),
                preferred_element_type=jnp.float32,
                precision=jax.lax.Precision.HIGHEST)
            d2 = sq_r + sq_all - 2.0 * g
            rows = t * _RT + jax.lax.broadcasted_iota(jnp.int32, (_RT, _N), 0)
            cols = jax.lax.broadcasted_iota(jnp.int32, (_RT, _N), 1)
            d2 = jnp.where(rows == cols, jnp.inf, jnp.maximum(d2, 1e-12))
            d2_ref[...] = d2
            sum_log = jnp.zeros((_RT, 1), jnp.float32)
            m = jnp.zeros((_RT, 1), jnp.float32)
            for _ in range(_K - 1):
                d2 = d2_ref[...]
                m = jnp.min(d2, axis=1, keepdims=True)
                sum_log = sum_log + jnp.log(m)
                d2_ref[...] = jnp.where(d2 <= m, jnp.inf, d2)
            s = 0.5 * ((_K - 1) * jnp.log(m) - sum_log)
            return total + jnp.sum(s)

        total = jax.lax.fori_loop(0, _NT, tile_step, 0.0)
        out_ref[b] = (_K - 2) * _N / total


def kernel(X):
    xt = jnp.swapaxes(X, 1, 2)
    return pl.pallas_call(
        _dim_body,
        out_shape=jax.ShapeDtypeStruct((_B,), jnp.float32),
        out_specs=pl.BlockSpec(memory_space=pltpu.SMEM),
        scratch_shapes=[pltpu.VMEM((_RT, _N), jnp.float32)],
    )(X, xt)
